# baseline (speedup 1.0000x reference)
import jax
import jax.numpy as jnp
from jax.experimental import pallas as pl
from jax.experimental.pallas import tpu as pltpu

TILE = 4096
NG = TILE // 512

F32 = jnp.float32


def _stacked_weights(w1, b1, w2, b2, w3, b3, w4, b4):
    w1s = jnp.zeros((128, 16), F32)
    for c in range(4):
        for k in range(3):
            w1s = w1s.at[32 * c:32 * c + 32, 4 * k + c].set(w1[:, k])
        w1s = w1s.at[32 * c:32 * c + 32, 12].set(b1[:, 0])
        w1s = w1s.at[32 * c + 20, 12].set(1.0)

    def bd(w, b):
        m = jnp.zeros((128, 128), F32)
        for c in range(4):
            m = m.at[32 * c:32 * c + 32, 32 * c:32 * c + 32].set(w)
            m = m.at[32 * c:32 * c + 32, 32 * c + 20].set(b[:, 0])
            m = m.at[32 * c + 20, 32 * c + 20].set(1.0)
        return m

    w2bd = bd(w2, b2)
    w3bd = bd(w3, b3)

    w4s = jnp.zeros((8, 128), F32)
    for c in range(4):
        w4s = w4s.at[c, 32 * c:32 * c + 32].set(w4[0, :])
        w4s = w4s.at[c, 32 * c + 20].set(b4[0, 0])

    cast = lambda a: a.astype(jnp.bfloat16)
    return cast(w1s), cast(w2bd), cast(w3bd), cast(w4s)


def _mlp_kernel(x_ref, w1s_ref, w2bd_ref, w3bd_ref, w4s_ref, o_ref):
    xb = x_ref[...].astype(F32)
    xs = [xb[k, :].reshape(32, 128) for k in range(3)]

    ones4 = jnp.ones((4, 128), F32)
    xs_w = jnp.concatenate(
        [jnp.concatenate([xs[0][4 * g:4 * g + 4], xs[1][4 * g:4 * g + 4],
                          xs[2][4 * g:4 * g + 4], ones4], axis=0)
         for g in range(NG)], axis=1)

    bf = jnp.bfloat16
    h = jnp.dot(w1s_ref[...], xs_w.astype(bf), preferred_element_type=F32)
    h = jnp.maximum(h, 0.0)
    h = jnp.dot(w2bd_ref[...], h.astype(bf), preferred_element_type=F32)
    h = jnp.maximum(h, 0.0)
    h = jnp.dot(w3bd_ref[...], h.astype(bf), preferred_element_type=F32)
    h = jnp.maximum(h, 0.0)
    o4 = jnp.dot(w4s_ref[...], h.astype(bf), preferred_element_type=F32)

    o_ref[...] = jnp.concatenate(
        [o4[0:4, 128 * g:128 * g + 128] for g in range(NG)], axis=0)


@jax.jit
def _forward(x, w1, b1, w2, b2, w3, b3, w4, b4):
    n = x.shape[0]
    w1s, w2bd, w3bd, w4s = _stacked_weights(w1, b1, w2, b2, w3, b3, w4, b4)
    x_t = x.astype(jnp.bfloat16).T
    wspec = [pl.BlockSpec(w.shape, lambda i: (0, 0))
             for w in (w1s, w2bd, w3bd, w4s)]
    out_r = pl.pallas_call(
        _mlp_kernel,
        out_shape=jax.ShapeDtypeStruct((n // 128, 128), F32),
        grid=(n // TILE,),
        in_specs=[pl.BlockSpec((3, TILE), lambda i: (0, i))] + wspec,
        out_specs=pl.BlockSpec((TILE // 128, 128), lambda i: (i, 0)),
        compiler_params=pltpu.CompilerParams(
            dimension_semantics=("parallel",),
            vmem_limit_bytes=64 * 1024 * 1024,
        ),
    )(x_t, w1s, w2bd, w3bd, w4s)
    return jnp.reshape(out_r, (n, 1))


def kernel(x, w1, b1, w2, b2, w3, b3, w4, b4):
    return _forward(x, w1, b1, w2, b2, w3, b3, w4, b4)


# speedup vs baseline: 2.2230x; 2.2230x over previous
import jax
import jax.numpy as jnp
from jax.experimental import pallas as pl
from jax.experimental.pallas import tpu as pltpu

TILE = 32768
NG = TILE // 512

F32 = jnp.float32


def _stacked_weights(w1, b1, w2, b2, w3, b3, w4, b4):
    w1s = jnp.zeros((128, 16), F32)
    for c in range(4):
        for k in range(3):
            w1s = w1s.at[32 * c:32 * c + 32, 4 * k + c].set(w1[:, k])
        w1s = w1s.at[32 * c:32 * c + 32, 12].set(b1[:, 0])
        w1s = w1s.at[32 * c + 20, 12].set(1.0)

    def bd(w, b):
        m = jnp.zeros((128, 128), F32)
        for c in range(4):
            m = m.at[32 * c:32 * c + 32, 32 * c:32 * c + 32].set(w)
            m = m.at[32 * c:32 * c + 32, 32 * c + 20].set(b[:, 0])
            m = m.at[32 * c + 20, 32 * c + 20].set(1.0)
        return m

    w2bd = bd(w2, b2)
    w3bd = bd(w3, b3)

    w4s = jnp.zeros((8, 128), F32)
    for c in range(4):
        w4s = w4s.at[c, 32 * c:32 * c + 32].set(w4[0, :])
        w4s = w4s.at[c, 32 * c + 20].set(b4[0, 0])

    cast = lambda a: a.astype(jnp.bfloat16)
    return cast(w1s), cast(w2bd), cast(w3bd), cast(w4s)


def _mlp_kernel(x_ref, w1s_ref, w2bd_ref, w3bd_ref, w4s_ref, o_ref):
    xs = [x_ref[k, :].reshape(TILE // 128, 128) for k in range(3)]

    ones4 = jnp.ones((4, 128), F32)
    xs_w = jnp.concatenate(
        [jnp.concatenate([xs[0][4 * g:4 * g + 4], xs[1][4 * g:4 * g + 4],
                          xs[2][4 * g:4 * g + 4], ones4], axis=0)
         for g in range(NG)], axis=1)

    bf = jnp.bfloat16
    h = jnp.dot(w1s_ref[...], xs_w.astype(bf), preferred_element_type=F32)
    h = jnp.maximum(h, 0.0)
    h = jnp.dot(w2bd_ref[...], h.astype(bf), preferred_element_type=F32)
    h = jnp.maximum(h, 0.0)
    h = jnp.dot(w3bd_ref[...], h.astype(bf), preferred_element_type=F32)
    h = jnp.maximum(h, 0.0)
    o4 = jnp.dot(w4s_ref[...], h.astype(bf), preferred_element_type=F32)

    o_ref[...] = jnp.concatenate(
        [o4[0:4, 128 * g:128 * g + 128] for g in range(NG)], axis=0)


@jax.jit
def _forward(x, w1, b1, w2, b2, w3, b3, w4, b4):
    n = x.shape[0]
    w1s, w2bd, w3bd, w4s = _stacked_weights(w1, b1, w2, b2, w3, b3, w4, b4)
    x_t = x.T
    wspec = [pl.BlockSpec(w.shape, lambda i: (0, 0))
             for w in (w1s, w2bd, w3bd, w4s)]
    out_r = pl.pallas_call(
        _mlp_kernel,
        out_shape=jax.ShapeDtypeStruct((n // 128, 128), F32),
        grid=(n // TILE,),
        in_specs=[pl.BlockSpec((3, TILE), lambda i: (0, i))] + wspec,
        out_specs=pl.BlockSpec((TILE // 128, 128), lambda i: (i, 0)),
        compiler_params=pltpu.CompilerParams(
            dimension_semantics=("parallel",),
            vmem_limit_bytes=64 * 1024 * 1024,
        ),
    )(x_t, w1s, w2bd, w3bd, w4s)
    return jnp.reshape(out_r, (n, 1))


def kernel(x, w1, b1, w2, b2, w3, b3, w4, b4):
    return _forward(x, w1, b1, w2, b2, w3, b3, w4, b4)


# speedup vs baseline: 2.2632x; 1.0181x over previous
import jax
import jax.numpy as jnp
from jax.experimental import pallas as pl
from jax.experimental.pallas import tpu as pltpu

TILE = 65536
NG = TILE // 512

F32 = jnp.float32


def _stacked_weights(w1, b1, w2, b2, w3, b3, w4, b4):
    w1s = jnp.zeros((128, 16), F32)
    for c in range(4):
        for k in range(3):
            w1s = w1s.at[32 * c:32 * c + 32, 4 * k + c].set(w1[:, k])
        w1s = w1s.at[32 * c:32 * c + 32, 12].set(b1[:, 0])
        w1s = w1s.at[32 * c + 20, 12].set(1.0)

    def bd(w, b):
        m = jnp.zeros((128, 128), F32)
        for c in range(4):
            m = m.at[32 * c:32 * c + 32, 32 * c:32 * c + 32].set(w)
            m = m.at[32 * c:32 * c + 32, 32 * c + 20].set(b[:, 0])
            m = m.at[32 * c + 20, 32 * c + 20].set(1.0)
        return m

    w2bd = bd(w2, b2)
    w3bd = bd(w3, b3)

    w4s = jnp.zeros((8, 128), F32)
    for c in range(4):
        w4s = w4s.at[c, 32 * c:32 * c + 32].set(w4[0, :])
        w4s = w4s.at[c, 32 * c + 20].set(b4[0, 0])

    cast = lambda a: a.astype(jnp.bfloat16)
    return cast(w1s), cast(w2bd), cast(w3bd), cast(w4s)


def _mlp_kernel(x_ref, w1s_ref, w2bd_ref, w3bd_ref, w4s_ref, o_ref):
    xs = [x_ref[k, :].reshape(TILE // 128, 128) for k in range(3)]

    ones4 = jnp.ones((4, 128), F32)
    xs_w = jnp.concatenate(
        [jnp.concatenate([xs[0][4 * g:4 * g + 4], xs[1][4 * g:4 * g + 4],
                          xs[2][4 * g:4 * g + 4], ones4], axis=0)
         for g in range(NG)], axis=1)

    bf = jnp.bfloat16
    h = jnp.dot(w1s_ref[...], xs_w.astype(bf), preferred_element_type=F32)
    h = jnp.maximum(h, 0.0)
    h = jnp.dot(w2bd_ref[...], h.astype(bf), preferred_element_type=F32)
    h = jnp.maximum(h, 0.0)
    h = jnp.dot(w3bd_ref[...], h.astype(bf), preferred_element_type=F32)
    h = jnp.maximum(h, 0.0)
    o4 = jnp.dot(w4s_ref[...], h.astype(bf), preferred_element_type=F32)

    o_ref[...] = jnp.concatenate(
        [o4[0:4, 128 * g:128 * g + 128] for g in range(NG)], axis=0)


@jax.jit
def _forward(x, w1, b1, w2, b2, w3, b3, w4, b4):
    n = x.shape[0]
    w1s, w2bd, w3bd, w4s = _stacked_weights(w1, b1, w2, b2, w3, b3, w4, b4)
    x_t = x.T
    wspec = [pl.BlockSpec(w.shape, lambda i: (0, 0))
             for w in (w1s, w2bd, w3bd, w4s)]
    out_r = pl.pallas_call(
        _mlp_kernel,
        out_shape=jax.ShapeDtypeStruct((n // 128, 128), F32),
        grid=(n // TILE,),
        in_specs=[pl.BlockSpec((3, TILE), lambda i: (0, i))] + wspec,
        out_specs=pl.BlockSpec((TILE // 128, 128), lambda i: (i, 0)),
        compiler_params=pltpu.CompilerParams(
            dimension_semantics=("parallel",),
            vmem_limit_bytes=100 * 1024 * 1024,
        ),
    )(x_t, w1s, w2bd, w3bd, w4s)
    return jnp.reshape(out_r, (n, 1))


def kernel(x, w1, b1, w2, b2, w3, b3, w4, b4):
    return _forward(x, w1, b1, w2, b2, w3, b3, w4, b4)
